# Initial kernel scaffold; baseline (speedup 1.0000x reference)
import jax
import jax.numpy as jnp
from jax.experimental import pallas as pl


def kernel(x, edge_index, edge_attr, conv0_W, conv0_b, conv1_W, conv1_b, conv2_W, conv2_b, fc1_W, fc1_b, fc2_W, fc2_b, fc4_W, fc4_b, fc5_W, fc5_b):
    raise NotImplementedError("write your pallas kernel here")


# speedup vs baseline: 3.0997x; 3.0997x over previous
import functools

import jax
import jax.numpy as jnp
from jax import lax
from jax.experimental import pallas as pl
from jax.experimental.pallas import tpu as pltpu
from jax.experimental.pallas import tpu_sc as plsc

N = 10000
NP = 10240
HALF = NP // 2
D = 128
OUT = 2
NC = 2
NS = 16
NW = NC * NS
L = 16
E = 320000
CB = 128
NCH = 79
EPT = NCH * CB
EP = NW * EPT
NCHS = 2 * NCH
EPS = NCHS * CB
RPS = HALF // NS

_mesh = plsc.VectorSubcoreMesh(core_axis_name="c", subcore_axis_name="s")
_sc_params = pltpu.CompilerParams(needs_layout_passes=False)


def _wid():
    c = lax.axis_index("c")
    s = lax.axis_index("s")
    return c, s, c * NS + s


@functools.partial(
    pl.kernel,
    out_type=jax.ShapeDtypeStruct((NC, NP), jnp.float32),
    mesh=_mesh,
    compiler_params=_sc_params,
    scratch_types=[
        pltpu.VMEM((NCH, CB), jnp.int32),
        pltpu.VMEM((NCH, CB), jnp.float32),
        pltpu.VMEM((NP,), jnp.float32),
        pltpu.VMEM((NP // NS,), jnp.float32),
        pltpu.VMEM((NP // NS,), jnp.float32),
        pltpu.VMEM_SHARED((NS, NP), jnp.float32),
    ],
)
def _deg_kernel(col3, ew3, degp, col_v, ew_v, deg_v, tmp_v, sum_v, shared):
    c, s, w = _wid()
    pltpu.sync_copy(col3.at[w], col_v)
    pltpu.sync_copy(ew3.at[w], ew_v)
    zero = jnp.zeros((L,), jnp.float32)
    rpt = NP // NS

    def zbody(i, _):
        deg_v[pl.ds(i * L, L)] = zero
        return 0

    lax.fori_loop(0, NP // L, zbody, 0)

    def ebody(j, _):
        for g in range(CB // L):
            idx = col_v[j, pl.ds(g * L, L)]
            vals = ew_v[j, pl.ds(g * L, L)]
            plsc.addupdate_scatter(deg_v, [idx], vals)
        return 0

    lax.fori_loop(0, NCH, ebody, 0)
    pltpu.sync_copy(deg_v, shared.at[s])
    plsc.subcore_barrier()
    base = s * rpt

    def z2(i, _):
        sum_v[pl.ds(i * L, L)] = zero
        return 0

    lax.fori_loop(0, rpt // L, z2, 0)
    for t in range(NS):
        pltpu.sync_copy(shared.at[t, pl.ds(base, rpt)], tmp_v)

        def abody(i, _):
            sum_v[pl.ds(i * L, L)] = sum_v[pl.ds(i * L, L)] + tmp_v[pl.ds(i * L, L)]
            return 0

        lax.fori_loop(0, rpt // L, abody, 0)
    pltpu.sync_copy(sum_v, degp.at[c, pl.ds(base, rpt)])


@functools.partial(
    pl.kernel,
    out_type=(jax.ShapeDtypeStruct((NC, NW, NCH, CB), jnp.float32),
              jax.ShapeDtypeStruct((NC, NW, NCH, CB), jnp.int32)),
    mesh=_mesh,
    compiler_params=_sc_params,
    scratch_types=[
        pltpu.VMEM((NCH, CB), jnp.int32),
        pltpu.VMEM((NCH, CB), jnp.int32),
        pltpu.VMEM((NCH, CB), jnp.float32),
        pltpu.VMEM((NCH, CB), jnp.float32),
        pltpu.VMEM((NCH, CB), jnp.int32),
        pltpu.VMEM((NP,), jnp.float32),
        pltpu.VMEM((NP,), jnp.float32),
    ],
)
def _norm_kernel(degp, row3, col3, ew3, norm_sc, col_sc, row_v, col_v, ew_v,
                 nsc_v, csc_v, dinv_v, tmp_v):
    c, s, w = _wid()
    pltpu.sync_copy(row3.at[w], row_v)
    pltpu.sync_copy(col3.at[w], col_v)
    pltpu.sync_copy(ew3.at[w], ew_v)
    pltpu.sync_copy(degp.at[0], dinv_v)
    pltpu.sync_copy(degp.at[1], tmp_v)

    def dbody(i, _):
        d = dinv_v[pl.ds(i * L, L)] + tmp_v[pl.ds(i * L, L)]
        ib = plsc.bitcast(d, jnp.int32)
        y = plsc.bitcast(jnp.int32(0x5F3759DF) - (ib >> 1), jnp.float32)
        for _ in range(3):
            y = y * (1.5 - 0.5 * d * y * y)
        dinv_v[pl.ds(i * L, L)] = jnp.where(d > 0.0, y, 0.0)
        return 0

    lax.fori_loop(0, NP // L, dbody, 0)

    for cvar in range(NC):
        lo = cvar * HALF

        def nbody(j, _):
            for g in range(CB // L):
                sl = pl.ds(g * L, L)
                r = row_v[j, sl]
                cc = col_v[j, sl]
                dr = plsc.load_gather(dinv_v, [r])
                dc = plsc.load_gather(dinv_v, [cc])
                nm = dr * ew_v[j, sl] * dc
                rel = cc - lo
                ok = (rel >= 0) & (rel < HALF)
                nsc_v[j, sl] = jnp.where(ok, nm, 0.0)
                csc_v[j, sl] = jnp.clip(rel, 0, HALF - 1)
            return 0

        lax.fori_loop(0, NCH, nbody, 0)
        pltpu.sync_copy(nsc_v, norm_sc.at[cvar, w])
        pltpu.sync_copy(csc_v, col_sc.at[cvar, w])


@functools.partial(
    pl.kernel,
    out_type=jax.ShapeDtypeStruct((NP, D), jnp.float32),
    mesh=_mesh,
    compiler_params=_sc_params,
    scratch_types=[
        pltpu.VMEM((NCHS, CB), jnp.int32),
        pltpu.VMEM((NCHS, CB), jnp.int32),
        pltpu.VMEM((NCHS, CB), jnp.float32),
        pltpu.VMEM((CB, D), jnp.float32),
        pltpu.VMEM((RPS // 5, D), jnp.float32),
        pltpu.VMEM_SHARED((HALF, D), jnp.float32),
        pltpu.SemaphoreType.DMA,
    ],
)
def _prop_kernel(src, row3s, col_sc4, norm_sc4, out, row_v, col_v, norm_v,
                 buf, zbuf, acc, sem):
    c, s, _ = _wid()
    pltpu.sync_copy(row3s.at[s], row_v)
    pltpu.sync_copy(col_sc4.at[c, s], col_v)
    pltpu.sync_copy(norm_sc4.at[c, s], norm_v)
    zero = jnp.zeros((L,), jnp.float32)

    def zrow(i, _):
        for g in range(D // L):
            zbuf[i, pl.ds(g * L, L)] = zero
        return 0

    zb = RPS // 5
    lax.fori_loop(0, zb, zrow, 0)
    base = s * RPS
    for k in range(5):
        pltpu.sync_copy(zbuf, acc.at[pl.ds(base + k * zb, zb)])
    plsc.subcore_barrier()

    def chunk(j, _):
        pltpu.async_copy(src.at[row_v.at[j]], buf, sem).wait()

        def scale(rg, _):
            nv = norm_v[j, pl.ds(rg * L, L)]
            for e in range(L):
                r = rg * L + e
                sval = nv[e]
                for g in range(D // L):
                    sl = pl.ds(g * L, L)
                    buf[r, sl] = buf[r, sl] * sval
            return 0

        lax.fori_loop(0, CB // L, scale, 0)
        pltpu.sync_copy(buf, acc.at[col_v.at[j]], add=True)
        return 0

    lax.fori_loop(0, NCHS, chunk, 0)
    plsc.subcore_barrier()
    pltpu.sync_copy(acc.at[pl.ds(base, RPS)],
                    out.at[pl.ds(c * HALF + base, RPS)])


BR = 1024


def _layer_body(relu, h0, h1, h2, h3, W, b, o):
    acc = jnp.dot(h0[...], W[0], preferred_element_type=jnp.float32)
    acc = acc + jnp.dot(h1[...], W[1], preferred_element_type=jnp.float32)
    acc = acc + jnp.dot(h2[...], W[2], preferred_element_type=jnp.float32)
    acc = acc + jnp.dot(h3[...], W[3], preferred_element_type=jnp.float32)
    acc = acc + b[...]
    o[...] = jnp.maximum(acc, 0.0) if relu else acc


def _layer_call(relu, h0, h1, h2, h3, W, b):
    rows = pl.BlockSpec((BR, D), lambda i: (i, 0))
    return pl.pallas_call(
        functools.partial(_layer_body, relu),
        grid=(NP // BR,),
        in_specs=[rows] * 4 + [
            pl.BlockSpec((4, D, D), lambda i: (0, 0, 0)),
            pl.BlockSpec((1, D), lambda i: (0, 0)),
        ],
        out_specs=rows,
        out_shape=jax.ShapeDtypeStruct((NP, D), jnp.float32),
    )(h0, h1, h2, h3, W, b)


def _fc_body(h, w1, b1, w2, b2, w4, b4, w5, b5, o):
    a = jnp.maximum(jnp.dot(h[...], w1[...],
                            preferred_element_type=jnp.float32) + b1[...], 0.0)
    a = jnp.maximum(jnp.dot(a, w2[...],
                            preferred_element_type=jnp.float32) + b2[...], 0.0)
    a = jnp.maximum(jnp.dot(a, w4[...],
                            preferred_element_type=jnp.float32) + b4[...], 0.0)
    a = jnp.maximum(jnp.dot(a, w4[...],
                            preferred_element_type=jnp.float32) + b4[...], 0.0)
    o[...] = jnp.dot(a, w5[...], preferred_element_type=jnp.float32) + b5[...]


def _fc_call(h, w1, b1, w2, b2, w4, b4, w5, b5):
    rows = pl.BlockSpec((BR, D), lambda i: (i, 0))
    wspec = pl.BlockSpec((D, D), lambda i: (0, 0))
    bspec = pl.BlockSpec((1, D), lambda i: (0, 0))
    return pl.pallas_call(
        _fc_body,
        grid=(NP // BR,),
        in_specs=[rows, wspec, bspec, wspec, bspec, wspec, bspec, wspec,
                  bspec],
        out_specs=rows,
        out_shape=jax.ShapeDtypeStruct((NP, D), jnp.float32),
    )(h, w1, b1, w2, b2, w4, b4, w5, b5)


def kernel(x, edge_index, edge_attr, conv0_W, conv0_b, conv1_W, conv1_b,
           conv2_W, conv2_b, fc1_W, fc1_b, fc2_W, fc2_b, fc4_W, fc4_b,
           fc5_W, fc5_b):
    row = edge_index[0]
    col = edge_index[1]
    ew = edge_attr[:, 0]
    pad = EP - E
    row3 = jnp.pad(row, (0, pad)).reshape(NW, NCH, CB)
    col3 = jnp.pad(col, (0, pad)).reshape(NW, NCH, CB)
    ew3 = jnp.pad(ew, (0, pad)).reshape(NW, NCH, CB)
    x_pad = jnp.pad(x, ((0, NP - N), (0, 0)))

    degp = _deg_kernel(col3, ew3)
    norm_sc, col_sc = _norm_kernel(degp, row3, col3, ew3)
    row3s = row3.reshape(NS, NCHS, CB)
    col_sc4 = col_sc.reshape(NC, NS, NCHS, CB)
    norm_sc4 = norm_sc.reshape(NC, NS, NCHS, CB)

    h = x_pad
    for (W, b, relu) in ((conv0_W, conv0_b, True), (conv1_W, conv1_b, True),
                         (conv2_W, conv2_b, False)):
        h1 = _prop_kernel(h, row3s, col_sc4, norm_sc4)
        h2 = _prop_kernel(h1, row3s, col_sc4, norm_sc4)
        h3 = _prop_kernel(h2, row3s, col_sc4, norm_sc4)
        h = _layer_call(relu, h, h1, h2, h3, W, b.reshape(1, D))
    w5p = jnp.pad(fc5_W, ((0, 0), (0, D - OUT)))
    b5p = jnp.pad(fc5_b, (0, D - OUT)).reshape(1, D)
    out = _fc_call(h, fc1_W, fc1_b.reshape(1, D), fc2_W, fc2_b.reshape(1, D),
                   fc4_W, fc4_b.reshape(1, D), w5p, b5p)
    return out[:N, :OUT]
